# baseline (speedup 1.0000x reference)
import jax
import jax.numpy as jnp
from jax import lax
from jax.experimental import pallas as pl
from jax.experimental.pallas import tpu as pltpu
from jax.experimental.pallas import tpu_sc as plsc

N0, E0 = 10000, 320000
N1, E1 = 2500, 40000
D = 128
NC, NS = 2, 16
NW = NC * NS

ND0 = 10112
N1P = 2560
EPT0 = 10240
EPT1 = 1280
C = 128
B0, J0 = 5, 16
J1 = 10
JI, CI = 4, 80

_SC_PARAMS = pltpu.CompilerParams(needs_layout_passes=False,
                                  use_tc_tiling_on_sc=False)


def _zero_vec16():
    return jnp.zeros((16,), jnp.float32)


def _zero_2d(ref, rows):
    def _zrow(r, _):
        for k in range(8):
            ref[r, pl.ds(k * 16, 16)] = _zero_vec16()
        return 0
    lax.fori_loop(0, rows, _zrow, 0)


def _zero_1d(ref, n):
    def _z(i, _):
        ref[pl.ds(i * 16, 16)] = _zero_vec16()
        return 0
    lax.fori_loop(0, n // 16, _z, 0)


def _sc_g0_body(x0h, se0h, de0h,
                p0o,
                acc0, sev0, sev1, dev0, dev1, bufa, bufb,
                gsa, gsb, ssa, ssb, isem):
    cid = lax.axis_index("c")
    sid = lax.axis_index("s")
    wid = cid * NS + sid

    sev = [sev0, sev1]
    dev = [dev0, dev1]
    bufs = [bufa, bufb]
    gsem = [gsa, gsb]
    ssem = [ssa, ssb]

    _zero_2d(bufa, C)
    for t in range(4):
        pltpu.sync_copy(bufa, acc0.at[pl.ds(sid * 632 + t * 128, 128)])
    pltpu.sync_copy(bufa.at[pl.ds(0, 120)],
                    acc0.at[pl.ds(sid * 632 + 512, 120)])
    plsc.subcore_barrier()

    pltpu.sync_copy(se0h.at[wid, 0], sev[0])
    pltpu.sync_copy(de0h.at[wid, 0], dev[0])

    gd = [None, None]
    sd = [None, None]
    idxd = []
    gd[0] = pltpu.async_copy(x0h.at[sev[0].at[0]], bufs[0], gsem[0])

    for b in range(B0):
        ib = b % 2
        if b + 1 < B0:
            jprev = b * J0 - 1
            if jprev >= 0 and sd[jprev % 2] is not None:
                sd[jprev % 2].wait()
                sd[jprev % 2] = None
            nb = 1 - ib
            idxd = [pltpu.async_copy(se0h.at[wid, b + 1], sev[nb], isem),
                    pltpu.async_copy(de0h.at[wid, b + 1], dev[nb], isem)]
        for r in range(J0):
            j = b * J0 + r
            k = j % 2
            gd[k].wait()
            gd[k] = None
            if sd[1 - k] is not None:
                sd[1 - k].wait()
                sd[1 - k] = None
            jj = j + 1
            if jj < B0 * J0:
                bb, rr = divmod(jj, J0)
                if bb != b and idxd:
                    for dsc in idxd:
                        dsc.wait()
                    idxd = []
                gd[1 - k] = pltpu.async_copy(
                    x0h.at[sev[bb % 2].at[rr]], bufs[1 - k], gsem[1 - k])
            sd[k] = pltpu.async_copy(
                bufs[k], acc0.at[dev[ib].at[r]], ssem[k], add=True)

    for k in range(2):
        if sd[k] is not None:
            sd[k].wait()

    plsc.subcore_barrier()
    pltpu.sync_copy(acc0.at[pl.ds(sid * 632, 632)],
                    p0o.at[cid, pl.ds(sid * 632, 632)])


def _sc_g1_body(x1h, se1h, de1h, d0ch,
                p1o, d0o, d1o,
                acc1, se1v, de1v, d0cv, bufa, bufb, deg0, deg1,
                gsa, gsb, ssa, ssb):
    cid = lax.axis_index("c")
    sid = lax.axis_index("s")
    wid = cid * NS + sid

    bufs = [bufa, bufb]
    gsem = [gsa, gsb]
    ssem = [ssa, ssb]

    _zero_2d(bufa, C)
    _zero_1d(deg0, ND0)
    _zero_1d(deg1, N1P)
    pltpu.sync_copy(bufa, acc1.at[pl.ds(sid * 160, 128)])
    pltpu.sync_copy(bufa.at[pl.ds(0, 32)], acc1.at[pl.ds(sid * 160 + 128, 32)])
    plsc.subcore_barrier()

    pltpu.sync_copy(se1h.at[wid], se1v)
    pltpu.sync_copy(de1h.at[wid], de1v)
    pltpu.sync_copy(d0ch.at[wid], d0cv)

    gd = [None, None]
    sd = [None, None]
    gd[0] = pltpu.async_copy(x1h.at[se1v.at[0]], bufs[0], gsem[0])
    for r in range(J1):
        k = r % 2
        gd[k].wait()
        gd[k] = None
        if sd[1 - k] is not None:
            sd[1 - k].wait()
            sd[1 - k] = None
        if r + 1 < J1:
            gd[1 - k] = pltpu.async_copy(
                x1h.at[se1v.at[r + 1]], bufs[1 - k], gsem[1 - k])
        sd[k] = pltpu.async_copy(
            bufs[k], acc1.at[de1v.at[r]], ssem[k], add=True)

    ones16 = jnp.ones((16,), jnp.float32)

    def _h0(t, _):
        r = t // 8
        g = t % 8
        plsc.addupdate_scatter(deg0, [d0cv[r, pl.ds(g * 16, 16)]], ones16)
        return 0
    lax.fori_loop(0, (EPT0 // C) * 8, _h0, 0)

    def _h1(t, _):
        r = t // 8
        g = t % 8
        plsc.addupdate_scatter(deg1, [de1v[r, pl.ds(g * 16, 16)]], ones16)
        return 0
    lax.fori_loop(0, J1 * 8, _h1, 0)

    for k in range(2):
        if sd[k] is not None:
            sd[k].wait()

    plsc.subcore_barrier()
    pltpu.sync_copy(acc1.at[pl.ds(sid * 160, 160)],
                    p1o.at[cid, pl.ds(sid * 160, 160)])
    pltpu.sync_copy(deg0, d0o.at[wid])
    pltpu.sync_copy(deg1, d1o.at[wid])


def _sc_inter_body(z0h, z1h, clch, go, c2o, cnto,
                   accc, clv, r0, r1, r2, r3, g0b, g1b, g2b, g3b, cnt,
                   rs0, rs1, rs2, rs3, gs0, gs1, gs2, gs3):
    cid = lax.axis_index("c")
    sid = lax.axis_index("s")
    wid = cid * NS + sid

    rbufs = [r0, r1, r2, r3]
    gbufs = [g0b, g1b, g2b, g3b]
    rsem = [rs0, rs1, rs2, rs3]
    gsem = [gs0, gs1, gs2, gs3]

    _zero_2d(r0, CI)
    _zero_1d(cnt, N1P)
    pltpu.sync_copy(r0, accc.at[pl.ds(sid * 160, 80)])
    pltpu.sync_copy(r0, accc.at[pl.ds(sid * 160 + 80, 80)])
    pltpu.sync_copy(clch.at[wid], clv)
    plsc.subcore_barrier()

    ones16 = jnp.ones((16,), jnp.float32)
    base0 = wid * (JI * CI)

    for j in range(JI):
        valid = base0 + j * CI < N0

        @pl.when(valid)
        def _(j=j):
            pltpu.async_copy(z0h.at[pl.ds(base0 + j * CI, CI)],
                             rbufs[j], rsem[j])
            pltpu.async_copy(z1h.at[clv.at[j]], gbufs[j], gsem[j])

    for j in range(JI):
        valid = base0 + j * CI < N0

        @pl.when(valid)
        def _(j=j):
            pltpu.make_async_copy(z0h.at[pl.ds(base0 + j * CI, CI)],
                                  rbufs[j], rsem[j]).wait()
            pltpu.sync_copy(rbufs[j], accc.at[clv.at[j]], add=True)
            for g in range(5):
                plsc.addupdate_scatter(cnt, [clv[j, pl.ds(g * 16, 16)]],
                                       ones16)

    for j in range(JI):
        valid = base0 + j * CI < N0

        @pl.when(valid)
        def _(j=j):
            pltpu.make_async_copy(z1h.at[clv.at[j]], gbufs[j],
                                  gsem[j]).wait()
            pltpu.sync_copy(gbufs[j], go.at[pl.ds(base0 + j * CI, CI)])

    plsc.subcore_barrier()
    pltpu.sync_copy(accc.at[pl.ds(sid * 160, 160)],
                    c2o.at[cid, pl.ds(sid * 160, 160)])
    pltpu.sync_copy(cnt, cnto.at[wid])


def _mm(a, w):
    return lax.dot_general(a, w, (((1,), (0,)), ((), ())),
                           preferred_element_type=jnp.float32)


def _lin_body(x_ref, w_ref, o_ref):
    o_ref[...] = _mm(x_ref[...], w_ref[...])


def _tc2a_body(p0_ref, d0_ref, xs_ref, wn_ref, wsi_ref, wni_ref,
               h_ref, s_ref, z_ref):
    a = (p0_ref[0] + p0_ref[1])[:N0]
    deg = jnp.sum(d0_ref[...], axis=0)[:N0]
    m = a / jnp.maximum(deg, 1.0)[:, None]
    h = jnp.maximum(xs_ref[...] + _mm(m, wn_ref[...]), 0.0)
    h_ref[...] = h
    s_ref[...] = _mm(h, wsi_ref[...])
    z_ref[...] = _mm(h, wni_ref[...])


def _tc2b_body(p1_ref, d1_ref, xs_ref, wn_ref, wsi_ref, wni_ref,
               h_ref, s_ref, z_ref):
    a = (p1_ref[0] + p1_ref[1])[:N1]
    deg = jnp.sum(d1_ref[...], axis=0)[:N1]
    m = a / jnp.maximum(deg, 1.0)[:, None]
    h = jnp.maximum(xs_ref[...] + _mm(m, wn_ref[...]), 0.0)
    h_ref[...] = h
    s_ref[...] = _mm(h, wsi_ref[...])
    z_ref[...] = _mm(h, wni_ref[...])


def _tc3a_body(x_ref, h_ref, s_ref, g_ref, o_ref):
    nz = jnp.maximum(s_ref[...] + g_ref[...], 0.0)
    o_ref[...] = x_ref[...] + 0.5 * (h_ref[...] + nz)


def _tc3b_body(x_ref, h_ref, s_ref, c2_ref, cnt_ref, o_ref):
    cnt = jnp.sum(cnt_ref[...], axis=0)[:N1]
    cs = (c2_ref[0] + c2_ref[1])[:N1]
    nz = jnp.maximum(s_ref[...] + cs / jnp.maximum(cnt, 1.0)[:, None], 0.0)
    o_ref[...] = x_ref[...] + 0.5 * (h_ref[...] + nz)


def kernel(x0, x1, edge_index0, edge_index1, inter_edge_index,
           W_self0, W_neigh0, W_self1, W_neigh1, W_self_i, W_neigh_i):
    f32 = jnp.float32
    i32 = jnp.int32

    pad0 = EPT0 - E0 // NW
    pad1 = EPT1 - E1 // NW
    s0 = jnp.pad(edge_index0[0].reshape(NW, E0 // NW), ((0, 0), (0, pad0)))
    d0junk = 10000 + (jnp.arange(pad0, dtype=i32) % (ND0 - N0))
    d0 = jnp.concatenate(
        [edge_index0[1].reshape(NW, E0 // NW),
         jnp.broadcast_to(d0junk, (NW, pad0))], axis=1)
    src0 = s0.reshape(NW, B0, J0, C)
    dst0 = d0.reshape(NW, B0, J0, C)
    dst0c = d0.reshape(NW, EPT0 // C, C)
    s1 = jnp.pad(edge_index1[0].reshape(NW, E1 // NW), ((0, 0), (0, pad1)))
    d1junk = N1 + (jnp.arange(pad1, dtype=i32) % (N1P - N1))
    d1 = jnp.concatenate(
        [edge_index1[1].reshape(NW, E1 // NW),
         jnp.broadcast_to(d1junk, (NW, pad1))], axis=1)
    src1 = s1.reshape(NW, J1, C)
    dst1 = d1.reshape(NW, J1, C)
    cluster = inter_edge_index[1, :N0] - N0
    clc = jnp.pad(cluster, (0, NW * JI * CI - N0),
                  constant_values=N1).reshape(NW, JI, CI)

    mesh = plsc.VectorSubcoreMesh(core_axis_name="c", subcore_axis_name="s")

    p0 = pl.kernel(
        _sc_g0_body,
        out_type=jax.ShapeDtypeStruct((NC, ND0, D), f32),
        mesh=mesh,
        scratch_types=[
            pltpu.VMEM_SHARED((ND0, D), f32),
            pltpu.VMEM((J0, C), i32),
            pltpu.VMEM((J0, C), i32),
            pltpu.VMEM((J0, C), i32),
            pltpu.VMEM((J0, C), i32),
            pltpu.VMEM((C, D), f32),
            pltpu.VMEM((C, D), f32),
        ] + [pltpu.SemaphoreType.DMA] * 5,
        compiler_params=_SC_PARAMS,
        name="sc_g0_agg",
    )(x0, src0, dst0)

    p1, d0p, d1p = pl.kernel(
        _sc_g1_body,
        out_type=[
            jax.ShapeDtypeStruct((NC, N1P, D), f32),
            jax.ShapeDtypeStruct((NW, ND0), f32),
            jax.ShapeDtypeStruct((NW, N1P), f32),
        ],
        mesh=mesh,
        scratch_types=[
            pltpu.VMEM_SHARED((N1P, D), f32),
            pltpu.VMEM((J1, C), i32),
            pltpu.VMEM((J1, C), i32),
            pltpu.VMEM((EPT0 // C, C), i32),
            pltpu.VMEM((C, D), f32),
            pltpu.VMEM((C, D), f32),
            pltpu.VMEM((ND0,), f32),
            pltpu.VMEM((N1P,), f32),
        ] + [pltpu.SemaphoreType.DMA] * 4,
        compiler_params=_SC_PARAMS,
        name="sc_g1_agg",
    )(x1, src1, dst1, dst0c)

    xs0 = pl.pallas_call(
        _lin_body,
        grid=(5,),
        in_specs=[pl.BlockSpec((2000, D), lambda i: (i, 0)),
                  pl.BlockSpec((D, D), lambda i: (0, 0))],
        out_specs=pl.BlockSpec((2000, D), lambda i: (i, 0)),
        out_shape=jax.ShapeDtypeStruct((N0, D), f32),
    )(x0, W_self0)
    xs1 = pl.pallas_call(
        _lin_body,
        out_shape=jax.ShapeDtypeStruct((N1, D), f32),
    )(x1, W_self1)

    h0, s0, z0 = pl.pallas_call(
        _tc2a_body,
        out_shape=[jax.ShapeDtypeStruct((N0, D), f32),
                   jax.ShapeDtypeStruct((N0, D), f32),
                   jax.ShapeDtypeStruct((N0, D), f32)],
        compiler_params=pltpu.CompilerParams(vmem_limit_bytes=100 * 1024 * 1024),
    )(p0, d0p, xs0, W_neigh0, W_self_i, W_neigh_i)

    h1, s1, z1 = pl.pallas_call(
        _tc2b_body,
        out_shape=[jax.ShapeDtypeStruct((N1, D), f32),
                   jax.ShapeDtypeStruct((N1, D), f32),
                   jax.ShapeDtypeStruct((N1, D), f32)],
    )(p1, d1p, xs1, W_neigh1, W_self_i, W_neigh_i)

    g, c2, cntp = pl.kernel(
        _sc_inter_body,
        out_type=[
            jax.ShapeDtypeStruct((N0, D), f32),
            jax.ShapeDtypeStruct((NC, N1P, D), f32),
            jax.ShapeDtypeStruct((NW, N1P), f32),
        ],
        mesh=mesh,
        scratch_types=[
            pltpu.VMEM_SHARED((N1P, D), f32),
            pltpu.VMEM((JI, CI), i32),
            pltpu.VMEM((CI, D), f32),
            pltpu.VMEM((CI, D), f32),
            pltpu.VMEM((CI, D), f32),
            pltpu.VMEM((CI, D), f32),
            pltpu.VMEM((CI, D), f32),
            pltpu.VMEM((CI, D), f32),
            pltpu.VMEM((CI, D), f32),
            pltpu.VMEM((CI, D), f32),
            pltpu.VMEM((N1P,), f32),
        ] + [pltpu.SemaphoreType.DMA] * 8,
        compiler_params=_SC_PARAMS,
        name="sc_inter",
    )(z0, z1, clc)

    out0 = pl.pallas_call(
        _tc3a_body,
        grid=(5,),
        in_specs=[pl.BlockSpec((2000, D), lambda i: (i, 0))] * 4,
        out_specs=pl.BlockSpec((2000, D), lambda i: (i, 0)),
        out_shape=jax.ShapeDtypeStruct((N0, D), f32),
    )(x0, h0, s0, g)

    out1 = pl.pallas_call(
        _tc3b_body,
        out_shape=jax.ShapeDtypeStruct((N1, D), f32),
    )(x1, h1, s1, c2, cntp)

    return (out0, out1)


# speedup vs baseline: 2.2824x; 2.2824x over previous
import jax
import jax.numpy as jnp
from jax import lax
from jax.experimental import pallas as pl
from jax.experimental.pallas import tpu as pltpu
from jax.experimental.pallas import tpu_sc as plsc

N0, E0 = 10000, 320000
N1, E1 = 2500, 40000
D = 128
NC, NS = 2, 16
NW = NC * NS

ND0 = 10016
N1P = 2560
EPT1 = 1280
C = 80
B0, J0 = 5, 25
J1 = 16
JI, CI = 4, 80

_SC_PARAMS = pltpu.CompilerParams(needs_layout_passes=False,
                                  use_tc_tiling_on_sc=False)


def _zero_vec16():
    return jnp.zeros((16,), jnp.float32)


def _zero_2d(ref, rows):
    def _zrow(r, _):
        for k in range(8):
            ref[r, pl.ds(k * 16, 16)] = _zero_vec16()
        return 0
    lax.fori_loop(0, rows, _zrow, 0)


def _zero_1d(ref, n):
    def _z(i, _):
        ref[pl.ds(i * 16, 16)] = _zero_vec16()
        return 0
    lax.fori_loop(0, n // 16, _z, 0)


def _sc_g0_body(x0h, se0h, de0h,
                p0o, d0o,
                acc0, sev0, sev1, dev0, dev1, b0, b1, b2, deg0,
                gs0, gs1, gs2, ss0, ss1, ss2, isem):
    cid = lax.axis_index("c")
    sid = lax.axis_index("s")
    wid = cid * NS + sid

    sev = [sev0, sev1]
    dev = [dev0, dev1]
    bufs = [b0, b1, b2]
    gsem = [gs0, gs1, gs2]
    ssem = [ss0, ss1, ss2]

    _zero_2d(b0, C)
    _zero_1d(deg0, ND0)
    for t in range(7):
        pltpu.sync_copy(b0, acc0.at[pl.ds(sid * 625 + t * 80, 80)])
    pltpu.sync_copy(b0.at[pl.ds(0, 65)], acc0.at[pl.ds(sid * 625 + 560, 65)])
    plsc.subcore_barrier()

    pltpu.sync_copy(se0h.at[wid, 0], sev[0])
    pltpu.sync_copy(de0h.at[wid, 0], dev[0])

    ones16 = jnp.ones((16,), jnp.float32)
    gd = [None, None, None]
    sd = [None, None, None]
    idxd = []
    gd[0] = pltpu.async_copy(x0h.at[sev[0].at[0]], bufs[0], gsem[0])
    gd[1] = pltpu.async_copy(x0h.at[sev[0].at[1]], bufs[1], gsem[1])

    for b in range(B0):
        ib = b % 2
        if b + 1 < B0:
            jprev = b * J0 - 1
            if jprev >= 0 and sd[jprev % 3] is not None:
                sd[jprev % 3].wait()
                sd[jprev % 3] = None
            nb = 1 - ib
            idxd = [pltpu.async_copy(se0h.at[wid, b + 1], sev[nb], isem),
                    pltpu.async_copy(de0h.at[wid, b + 1], dev[nb], isem)]
        for r in range(J0):
            j = b * J0 + r
            k = j % 3
            gd[k].wait()
            gd[k] = None
            k2 = (j + 2) % 3
            if sd[k2] is not None:
                sd[k2].wait()
                sd[k2] = None
            jj = j + 2
            if jj < B0 * J0:
                bb, rr = divmod(jj, J0)
                if bb != b and idxd:
                    for dsc in idxd:
                        dsc.wait()
                    idxd = []
                gd[k2] = pltpu.async_copy(
                    x0h.at[sev[bb % 2].at[rr]], bufs[k2], gsem[k2])
            sd[k] = pltpu.async_copy(
                bufs[k], acc0.at[dev[ib].at[r]], ssem[k], add=True)

        def _h(t, _, ib=ib):
            r = t // 5
            g = t % 5
            plsc.addupdate_scatter(deg0, [dev[ib][r, pl.ds(g * 16, 16)]],
                                   ones16)
            return 0
        lax.fori_loop(0, J0 * 5, _h, 0)

    for k in range(3):
        if sd[k] is not None:
            sd[k].wait()

    plsc.subcore_barrier()
    pltpu.sync_copy(acc0.at[pl.ds(sid * 625, 625)],
                    p0o.at[cid, pl.ds(sid * 625, 625)])
    pltpu.sync_copy(deg0, d0o.at[wid])


def _sc_g1_body(x1h, se1h, de1h,
                p1o, d1o,
                acc1, se1v, de1v, bufa, bufb, deg1,
                gsa, gsb, ssa, ssb):
    cid = lax.axis_index("c")
    sid = lax.axis_index("s")
    wid = cid * NS + sid

    bufs = [bufa, bufb]
    gsem = [gsa, gsb]
    ssem = [ssa, ssb]

    _zero_2d(bufa, C)
    _zero_1d(deg1, N1P)
    pltpu.sync_copy(bufa, acc1.at[pl.ds(sid * 160, 80)])
    pltpu.sync_copy(bufa, acc1.at[pl.ds(sid * 160 + 80, 80)])
    plsc.subcore_barrier()

    pltpu.sync_copy(se1h.at[wid], se1v)
    pltpu.sync_copy(de1h.at[wid], de1v)

    gd = [None, None]
    sd = [None, None]
    gd[0] = pltpu.async_copy(x1h.at[se1v.at[0]], bufs[0], gsem[0])
    for r in range(J1):
        k = r % 2
        gd[k].wait()
        gd[k] = None
        if sd[1 - k] is not None:
            sd[1 - k].wait()
            sd[1 - k] = None
        if r + 1 < J1:
            gd[1 - k] = pltpu.async_copy(
                x1h.at[se1v.at[r + 1]], bufs[1 - k], gsem[1 - k])
        sd[k] = pltpu.async_copy(
            bufs[k], acc1.at[de1v.at[r]], ssem[k], add=True)

    ones16 = jnp.ones((16,), jnp.float32)

    def _h1(t, _):
        r = t // 5
        g = t % 5
        plsc.addupdate_scatter(deg1, [de1v[r, pl.ds(g * 16, 16)]], ones16)
        return 0
    lax.fori_loop(0, J1 * 5, _h1, 0)

    for k in range(2):
        if sd[k] is not None:
            sd[k].wait()

    plsc.subcore_barrier()
    pltpu.sync_copy(acc1.at[pl.ds(sid * 160, 160)],
                    p1o.at[cid, pl.ds(sid * 160, 160)])
    pltpu.sync_copy(deg1, d1o.at[wid])


def _sc_inter_body(z0h, z1h, clch, go, c2o, cnto,
                   accc, clv, r0, r1, r2, r3, g0b, g1b, g2b, g3b, cnt,
                   rs0, rs1, rs2, rs3, gs0, gs1, gs2, gs3):
    cid = lax.axis_index("c")
    sid = lax.axis_index("s")
    wid = cid * NS + sid

    rbufs = [r0, r1, r2, r3]
    gbufs = [g0b, g1b, g2b, g3b]
    rsem = [rs0, rs1, rs2, rs3]
    gsem = [gs0, gs1, gs2, gs3]

    _zero_2d(r0, CI)
    _zero_1d(cnt, N1P)
    pltpu.sync_copy(r0, accc.at[pl.ds(sid * 160, 80)])
    pltpu.sync_copy(r0, accc.at[pl.ds(sid * 160 + 80, 80)])
    pltpu.sync_copy(clch.at[wid], clv)
    plsc.subcore_barrier()

    ones16 = jnp.ones((16,), jnp.float32)
    base0 = wid * (JI * CI)

    for j in range(JI):
        valid = base0 + j * CI < N0

        @pl.when(valid)
        def _(j=j):
            pltpu.async_copy(z0h.at[pl.ds(base0 + j * CI, CI)],
                             rbufs[j], rsem[j])
            pltpu.async_copy(z1h.at[clv.at[j]], gbufs[j], gsem[j])

    for j in range(JI):
        valid = base0 + j * CI < N0

        @pl.when(valid)
        def _(j=j):
            pltpu.make_async_copy(z0h.at[pl.ds(base0 + j * CI, CI)],
                                  rbufs[j], rsem[j]).wait()
            pltpu.sync_copy(rbufs[j], accc.at[clv.at[j]], add=True)
            for g in range(5):
                plsc.addupdate_scatter(cnt, [clv[j, pl.ds(g * 16, 16)]],
                                       ones16)

    for j in range(JI):
        valid = base0 + j * CI < N0

        @pl.when(valid)
        def _(j=j):
            pltpu.make_async_copy(z1h.at[clv.at[j]], gbufs[j],
                                  gsem[j]).wait()
            pltpu.sync_copy(gbufs[j], go.at[pl.ds(base0 + j * CI, CI)])

    plsc.subcore_barrier()
    pltpu.sync_copy(accc.at[pl.ds(sid * 160, 160)],
                    c2o.at[cid, pl.ds(sid * 160, 160)])
    pltpu.sync_copy(cnt, cnto.at[wid])


def _mm(a, w):
    return lax.dot_general(a, w, (((1,), (0,)), ((), ())),
                           preferred_element_type=jnp.float32)


def _lin_body(x_ref, w_ref, o_ref):
    o_ref[...] = _mm(x_ref[...], w_ref[...])


def _tc2a_body(p0_ref, d0_ref, xs_ref, wn_ref, wsi_ref, wni_ref,
               h_ref, s_ref, z_ref):
    a = p0_ref[0] + p0_ref[1]
    deg = jnp.sum(d0_ref[...], axis=0)[:N0]
    m = a / jnp.maximum(deg, 1.0)[:, None]
    h = jnp.maximum(xs_ref[...] + _mm(m, wn_ref[...]), 0.0)
    h_ref[...] = h
    s_ref[...] = _mm(h, wsi_ref[...])
    z_ref[...] = _mm(h, wni_ref[...])


def _tc2b_body(p1_ref, d1_ref, xs_ref, wn_ref, wsi_ref, wni_ref,
               h_ref, s_ref, z_ref):
    a = (p1_ref[0] + p1_ref[1])[:N1]
    deg = jnp.sum(d1_ref[...], axis=0)[:N1]
    m = a / jnp.maximum(deg, 1.0)[:, None]
    h = jnp.maximum(xs_ref[...] + _mm(m, wn_ref[...]), 0.0)
    h_ref[...] = h
    s_ref[...] = _mm(h, wsi_ref[...])
    z_ref[...] = _mm(h, wni_ref[...])


def _tc3a_body(x_ref, h_ref, s_ref, g_ref, o_ref):
    nz = jnp.maximum(s_ref[...] + g_ref[...], 0.0)
    o_ref[...] = x_ref[...] + 0.5 * (h_ref[...] + nz)


def _tc3b_body(x_ref, h_ref, s_ref, c2_ref, cnt_ref, o_ref):
    cnt = jnp.sum(cnt_ref[...], axis=0)[:N1]
    cs = (c2_ref[0] + c2_ref[1])[:N1]
    nz = jnp.maximum(s_ref[...] + cs / jnp.maximum(cnt, 1.0)[:, None], 0.0)
    o_ref[...] = x_ref[...] + 0.5 * (h_ref[...] + nz)


def kernel(x0, x1, edge_index0, edge_index1, inter_edge_index,
           W_self0, W_neigh0, W_self1, W_neigh1, W_self_i, W_neigh_i):
    f32 = jnp.float32
    i32 = jnp.int32

    src0 = edge_index0[0].reshape(NW, B0, J0, C)
    dst0 = edge_index0[1].reshape(NW, B0, J0, C)
    pad1 = EPT1 - E1 // NW
    s1 = jnp.pad(edge_index1[0].reshape(NW, E1 // NW), ((0, 0), (0, pad1)))
    d1junk = N1 + (jnp.arange(pad1, dtype=i32) % (N1P - N1))
    d1 = jnp.concatenate(
        [edge_index1[1].reshape(NW, E1 // NW),
         jnp.broadcast_to(d1junk, (NW, pad1))], axis=1)
    src1 = s1.reshape(NW, J1, C)
    dst1 = d1.reshape(NW, J1, C)
    cluster = inter_edge_index[1, :N0] - N0
    clc = jnp.pad(cluster, (0, NW * JI * CI - N0),
                  constant_values=N1).reshape(NW, JI, CI)

    mesh = plsc.VectorSubcoreMesh(core_axis_name="c", subcore_axis_name="s")

    p0, d0p = pl.kernel(
        _sc_g0_body,
        out_type=[
            jax.ShapeDtypeStruct((NC, N0, D), f32),
            jax.ShapeDtypeStruct((NW, ND0), f32),
        ],
        mesh=mesh,
        scratch_types=[
            pltpu.VMEM_SHARED((N0, D), f32),
            pltpu.VMEM((J0, C), i32),
            pltpu.VMEM((J0, C), i32),
            pltpu.VMEM((J0, C), i32),
            pltpu.VMEM((J0, C), i32),
            pltpu.VMEM((C, D), f32),
            pltpu.VMEM((C, D), f32),
            pltpu.VMEM((C, D), f32),
            pltpu.VMEM((ND0,), f32),
        ] + [pltpu.SemaphoreType.DMA] * 7,
        compiler_params=_SC_PARAMS,
        name="sc_g0_agg",
    )(x0, src0, dst0)

    p1, d1p = pl.kernel(
        _sc_g1_body,
        out_type=[
            jax.ShapeDtypeStruct((NC, N1P, D), f32),
            jax.ShapeDtypeStruct((NW, N1P), f32),
        ],
        mesh=mesh,
        scratch_types=[
            pltpu.VMEM_SHARED((N1P, D), f32),
            pltpu.VMEM((J1, C), i32),
            pltpu.VMEM((J1, C), i32),
            pltpu.VMEM((C, D), f32),
            pltpu.VMEM((C, D), f32),
            pltpu.VMEM((N1P,), f32),
        ] + [pltpu.SemaphoreType.DMA] * 4,
        compiler_params=_SC_PARAMS,
        name="sc_g1_agg",
    )(x1, src1, dst1)

    xs0 = pl.pallas_call(
        _lin_body,
        grid=(5,),
        in_specs=[pl.BlockSpec((2000, D), lambda i: (i, 0)),
                  pl.BlockSpec((D, D), lambda i: (0, 0))],
        out_specs=pl.BlockSpec((2000, D), lambda i: (i, 0)),
        out_shape=jax.ShapeDtypeStruct((N0, D), f32),
    )(x0, W_self0)
    xs1 = pl.pallas_call(
        _lin_body,
        out_shape=jax.ShapeDtypeStruct((N1, D), f32),
    )(x1, W_self1)

    h0, s0, z0 = pl.pallas_call(
        _tc2a_body,
        out_shape=[jax.ShapeDtypeStruct((N0, D), f32),
                   jax.ShapeDtypeStruct((N0, D), f32),
                   jax.ShapeDtypeStruct((N0, D), f32)],
        compiler_params=pltpu.CompilerParams(vmem_limit_bytes=100 * 1024 * 1024),
    )(p0, d0p, xs0, W_neigh0, W_self_i, W_neigh_i)

    h1, s1, z1 = pl.pallas_call(
        _tc2b_body,
        out_shape=[jax.ShapeDtypeStruct((N1, D), f32),
                   jax.ShapeDtypeStruct((N1, D), f32),
                   jax.ShapeDtypeStruct((N1, D), f32)],
    )(p1, d1p, xs1, W_neigh1, W_self_i, W_neigh_i)

    g, c2, cntp = pl.kernel(
        _sc_inter_body,
        out_type=[
            jax.ShapeDtypeStruct((N0, D), f32),
            jax.ShapeDtypeStruct((NC, N1P, D), f32),
            jax.ShapeDtypeStruct((NW, N1P), f32),
        ],
        mesh=mesh,
        scratch_types=[
            pltpu.VMEM_SHARED((N1P, D), f32),
            pltpu.VMEM((JI, CI), i32),
            pltpu.VMEM((CI, D), f32),
            pltpu.VMEM((CI, D), f32),
            pltpu.VMEM((CI, D), f32),
            pltpu.VMEM((CI, D), f32),
            pltpu.VMEM((CI, D), f32),
            pltpu.VMEM((CI, D), f32),
            pltpu.VMEM((CI, D), f32),
            pltpu.VMEM((CI, D), f32),
            pltpu.VMEM((N1P,), f32),
        ] + [pltpu.SemaphoreType.DMA] * 8,
        compiler_params=_SC_PARAMS,
        name="sc_inter",
    )(z0, z1, clc)

    out0 = pl.pallas_call(
        _tc3a_body,
        grid=(5,),
        in_specs=[pl.BlockSpec((2000, D), lambda i: (i, 0))] * 4,
        out_specs=pl.BlockSpec((2000, D), lambda i: (i, 0)),
        out_shape=jax.ShapeDtypeStruct((N0, D), f32),
    )(x0, h0, s0, g)

    out1 = pl.pallas_call(
        _tc3b_body,
        out_shape=jax.ShapeDtypeStruct((N1, D), f32),
    )(x1, h1, s1, c2, cntp)

    return (out0, out1)


# speedup vs baseline: 2.6481x; 1.1602x over previous
import jax
import jax.numpy as jnp
from jax import lax
from jax.experimental import pallas as pl
from jax.experimental.pallas import tpu as pltpu
from jax.experimental.pallas import tpu_sc as plsc

N0, E0 = 10000, 320000
N1, E1 = 2500, 40000
D = 128
NC, NS = 2, 16
NW = NC * NS

ND0 = 10112
N1P = 2560
EPT1 = 1280
C = 80
B0, J0 = 5, 25
J1 = 16
JI, CI = 4, 80

_SC_PARAMS = pltpu.CompilerParams(needs_layout_passes=False,
                                  use_tc_tiling_on_sc=False)


def _zero_vec16():
    return jnp.zeros((16,), jnp.float32)


def _zero_2d(ref, rows):
    def _zrow(r, _):
        for k in range(8):
            ref[r, pl.ds(k * 16, 16)] = _zero_vec16()
        return 0
    lax.fori_loop(0, rows, _zrow, 0)


def _zero_1d(ref, n):
    def _z(i, _):
        ref[pl.ds(i * 16, 16)] = _zero_vec16()
        return 0
    lax.fori_loop(0, n // 16, _z, 0)


def _sc_g0_body(x0h, se0h, de0h,
                p0o, d0o,
                acc0, sev0, sev1, dev0, dev1, b0, b1, b2, deg0,
                gs0, gs1, gs2, ss0, ss1, ss2, isem):
    cid = lax.axis_index("c")
    sid = lax.axis_index("s")
    wid = cid * NS + sid

    sev = [sev0, sev1]
    dev = [dev0, dev1]
    bufs = [b0, b1, b2]
    gsem = [gs0, gs1, gs2]
    ssem = [ss0, ss1, ss2]

    _zero_2d(b0, C)
    _zero_1d(deg0, ND0)
    for t in range(7):
        pltpu.sync_copy(b0, acc0.at[pl.ds(sid * 625 + t * 80, 80)])
    pltpu.sync_copy(b0.at[pl.ds(0, 65)], acc0.at[pl.ds(sid * 625 + 560, 65)])
    plsc.subcore_barrier()

    pltpu.sync_copy(se0h.at[wid, 0], sev[0])
    pltpu.sync_copy(de0h.at[wid, 0], dev[0])

    ones16 = jnp.ones((16,), jnp.float32)
    gd = [None, None, None]
    sd = [None, None, None]
    idxd = []
    gd[0] = pltpu.async_copy(x0h.at[sev[0].at[0]], bufs[0], gsem[0])
    gd[1] = pltpu.async_copy(x0h.at[sev[0].at[1]], bufs[1], gsem[1])

    for b in range(B0):
        ib = b % 2
        if b + 1 < B0:
            jprev = b * J0 - 1
            if jprev >= 0 and sd[jprev % 3] is not None:
                sd[jprev % 3].wait()
                sd[jprev % 3] = None
            nb = 1 - ib
            idxd = [pltpu.async_copy(se0h.at[wid, b + 1], sev[nb], isem),
                    pltpu.async_copy(de0h.at[wid, b + 1], dev[nb], isem)]
        for r in range(J0):
            j = b * J0 + r
            k = j % 3
            gd[k].wait()
            gd[k] = None
            k2 = (j + 2) % 3
            if sd[k2] is not None:
                sd[k2].wait()
                sd[k2] = None
            jj = j + 2
            if jj < B0 * J0:
                bb, rr = divmod(jj, J0)
                if bb != b and idxd:
                    for dsc in idxd:
                        dsc.wait()
                    idxd = []
                gd[k2] = pltpu.async_copy(
                    x0h.at[sev[bb % 2].at[rr]], bufs[k2], gsem[k2])
            sd[k] = pltpu.async_copy(
                bufs[k], acc0.at[dev[ib].at[r]], ssem[k], add=True)

        def _h(t, _, ib=ib):
            r = t // 5
            g = t % 5
            plsc.addupdate_scatter(deg0, [dev[ib][r, pl.ds(g * 16, 16)]],
                                   ones16)
            return 0
        lax.fori_loop(0, J0 * 5, _h, 0)

    for k in range(3):
        if sd[k] is not None:
            sd[k].wait()

    plsc.subcore_barrier()
    pltpu.sync_copy(acc0.at[pl.ds(sid * 625, 625)],
                    p0o.at[cid, pl.ds(sid * 625, 625)])
    pltpu.sync_copy(deg0, d0o.at[wid])


def _sc_g1_body(x1h, se1h, de1h,
                p1o, d1o,
                acc1, x1s, se1v, de1v, bufa, bufb, deg1,
                gsa, gsb, ssa, ssb):
    cid = lax.axis_index("c")
    sid = lax.axis_index("s")
    wid = cid * NS + sid

    bufs = [bufa, bufb]
    gsem = [gsa, gsb]
    ssem = [ssa, ssb]

    @pl.when(sid < 4)
    def _():
        pltpu.sync_copy(x1h.at[pl.ds(sid * 625, 625)],
                        x1s.at[pl.ds(sid * 625, 625)])

    _zero_2d(bufa, C)
    _zero_1d(deg1, N1P)
    pltpu.sync_copy(bufa, acc1.at[pl.ds(sid * 160, 80)])
    pltpu.sync_copy(bufa, acc1.at[pl.ds(sid * 160 + 80, 80)])
    pltpu.sync_copy(se1h.at[wid], se1v)
    pltpu.sync_copy(de1h.at[wid], de1v)
    plsc.subcore_barrier()

    gd = [None, None]
    sd = [None, None]
    gd[0] = pltpu.async_copy(x1s.at[se1v.at[0]], bufs[0], gsem[0])
    for r in range(J1):
        k = r % 2
        gd[k].wait()
        gd[k] = None
        if sd[1 - k] is not None:
            sd[1 - k].wait()
            sd[1 - k] = None
        if r + 1 < J1:
            gd[1 - k] = pltpu.async_copy(
                x1s.at[se1v.at[r + 1]], bufs[1 - k], gsem[1 - k])
        sd[k] = pltpu.async_copy(
            bufs[k], acc1.at[de1v.at[r]], ssem[k], add=True)

    ones16 = jnp.ones((16,), jnp.float32)

    def _h1(t, _):
        r = t // 5
        g = t % 5
        plsc.addupdate_scatter(deg1, [de1v[r, pl.ds(g * 16, 16)]], ones16)
        return 0
    lax.fori_loop(0, J1 * 5, _h1, 0)

    for k in range(2):
        if sd[k] is not None:
            sd[k].wait()

    plsc.subcore_barrier()
    pltpu.sync_copy(acc1.at[pl.ds(sid * 160, 160)],
                    p1o.at[cid, pl.ds(sid * 160, 160)])
    pltpu.sync_copy(deg1, d1o.at[wid])


def _sc_inter_body(z0h, z1h, clch, go, c2o, cnto,
                   accc, z1s, clv, r0, r1, r2, r3, g0b, g1b, g2b, g3b, cnt,
                   rs0, rs1, rs2, rs3, gs0, gs1, gs2, gs3):
    cid = lax.axis_index("c")
    sid = lax.axis_index("s")
    wid = cid * NS + sid

    rbufs = [r0, r1, r2, r3]
    gbufs = [g0b, g1b, g2b, g3b]
    rsem = [rs0, rs1, rs2, rs3]
    gsem = [gs0, gs1, gs2, gs3]

    @pl.when(sid < 4)
    def _():
        pltpu.sync_copy(z1h.at[pl.ds(sid * 625, 625)],
                        z1s.at[pl.ds(sid * 625, 625)])

    _zero_2d(r0, CI)
    _zero_1d(cnt, N1P)
    pltpu.sync_copy(r0, accc.at[pl.ds(sid * 160, 80)])
    pltpu.sync_copy(r0, accc.at[pl.ds(sid * 160 + 80, 80)])
    pltpu.sync_copy(clch.at[wid], clv)
    plsc.subcore_barrier()

    ones16 = jnp.ones((16,), jnp.float32)
    base0 = wid * (JI * CI)

    for j in range(JI):
        valid = base0 + j * CI < N0

        @pl.when(valid)
        def _(j=j):
            pltpu.async_copy(z0h.at[pl.ds(base0 + j * CI, CI)],
                             rbufs[j], rsem[j])
            pltpu.async_copy(z1s.at[clv.at[j]], gbufs[j], gsem[j])

    for j in range(JI):
        valid = base0 + j * CI < N0

        @pl.when(valid)
        def _(j=j):
            pltpu.make_async_copy(z0h.at[pl.ds(base0 + j * CI, CI)],
                                  rbufs[j], rsem[j]).wait()
            pltpu.sync_copy(rbufs[j], accc.at[clv.at[j]], add=True)
            for g in range(5):
                plsc.addupdate_scatter(cnt, [clv[j, pl.ds(g * 16, 16)]],
                                       ones16)

    for j in range(JI):
        valid = base0 + j * CI < N0

        @pl.when(valid)
        def _(j=j):
            pltpu.make_async_copy(z1s.at[clv.at[j]], gbufs[j],
                                  gsem[j]).wait()
            pltpu.sync_copy(gbufs[j], go.at[pl.ds(base0 + j * CI, CI)])

    plsc.subcore_barrier()
    pltpu.sync_copy(accc.at[pl.ds(sid * 160, 160)],
                    c2o.at[cid, pl.ds(sid * 160, 160)])
    pltpu.sync_copy(cnt, cnto.at[wid])


def _mm(a, w):
    return lax.dot_general(a, w, (((1,), (0,)), ((), ())),
                           preferred_element_type=jnp.float32)


def _lin_body(x_ref, w_ref, o_ref):
    o_ref[...] = _mm(x_ref[...], w_ref[...])


def _tc2a_body(p0_ref, d0_ref, xs_ref, wn_ref, wsi_ref, wni_ref,
               h_ref, s_ref, z_ref):
    a = p0_ref[0] + p0_ref[1]
    deg = jnp.sum(d0_ref[...], axis=0)[:N0]
    m = a / jnp.maximum(deg, 1.0)[:, None]
    h = jnp.maximum(xs_ref[...] + _mm(m, wn_ref[...]), 0.0)
    h_ref[...] = h
    s_ref[...] = _mm(h, wsi_ref[...])
    z_ref[...] = _mm(h, wni_ref[...])


def _tc2b_body(p1_ref, d1_ref, xs_ref, wn_ref, wsi_ref, wni_ref,
               h_ref, s_ref, z_ref):
    a = (p1_ref[0] + p1_ref[1])[:N1]
    deg = jnp.sum(d1_ref[...], axis=0)[:N1]
    m = a / jnp.maximum(deg, 1.0)[:, None]
    h = jnp.maximum(xs_ref[...] + _mm(m, wn_ref[...]), 0.0)
    h_ref[...] = h
    s_ref[...] = _mm(h, wsi_ref[...])
    z_ref[...] = _mm(h, wni_ref[...])


def _tc3a_body(x_ref, h_ref, s_ref, g_ref, o_ref):
    nz = jnp.maximum(s_ref[...] + g_ref[...], 0.0)
    o_ref[...] = x_ref[...] + 0.5 * (h_ref[...] + nz)


def _tc3b_body(x_ref, h_ref, s_ref, c2_ref, cnt_ref, o_ref):
    cnt = jnp.sum(cnt_ref[...], axis=0)[:N1]
    cs = (c2_ref[0] + c2_ref[1])[:N1]
    nz = jnp.maximum(s_ref[...] + cs / jnp.maximum(cnt, 1.0)[:, None], 0.0)
    o_ref[...] = x_ref[...] + 0.5 * (h_ref[...] + nz)


def kernel(x0, x1, edge_index0, edge_index1, inter_edge_index,
           W_self0, W_neigh0, W_self1, W_neigh1, W_self_i, W_neigh_i):
    f32 = jnp.float32
    i32 = jnp.int32

    src0 = edge_index0[0].reshape(NW, B0, J0, C)
    dst0 = edge_index0[1].reshape(NW, B0, J0, C)
    pad1 = EPT1 - E1 // NW
    s1 = jnp.pad(edge_index1[0].reshape(NW, E1 // NW), ((0, 0), (0, pad1)))
    d1junk = N1 + (jnp.arange(pad1, dtype=i32) % (N1P - N1))
    d1 = jnp.concatenate(
        [edge_index1[1].reshape(NW, E1 // NW),
         jnp.broadcast_to(d1junk, (NW, pad1))], axis=1)
    src1 = s1.reshape(NW, J1, C)
    dst1 = d1.reshape(NW, J1, C)
    cluster = inter_edge_index[1, :N0] - N0
    clc = jnp.pad(cluster, (0, NW * JI * CI - N0),
                  constant_values=N1).reshape(NW, JI, CI)

    mesh = plsc.VectorSubcoreMesh(core_axis_name="c", subcore_axis_name="s")

    p1, d1p = pl.kernel(
        _sc_g1_body,
        out_type=[
            jax.ShapeDtypeStruct((NC, N1P, D), f32),
            jax.ShapeDtypeStruct((NW, N1P), f32),
        ],
        mesh=mesh,
        scratch_types=[
            pltpu.VMEM_SHARED((N1P, D), f32),
            pltpu.VMEM_SHARED((N1, D), f32),
            pltpu.VMEM((J1, C), i32),
            pltpu.VMEM((J1, C), i32),
            pltpu.VMEM((C, D), f32),
            pltpu.VMEM((C, D), f32),
            pltpu.VMEM((N1P,), f32),
        ] + [pltpu.SemaphoreType.DMA] * 4,
        compiler_params=_SC_PARAMS,
        name="sc_g1_agg",
    )(x1, src1, dst1)

    p0, d0p = pl.kernel(
        _sc_g0_body,
        out_type=[
            jax.ShapeDtypeStruct((NC, N0, D), f32),
            jax.ShapeDtypeStruct((NW, ND0), f32),
        ],
        mesh=mesh,
        scratch_types=[
            pltpu.VMEM_SHARED((N0, D), f32),
            pltpu.VMEM((J0, C), i32),
            pltpu.VMEM((J0, C), i32),
            pltpu.VMEM((J0, C), i32),
            pltpu.VMEM((J0, C), i32),
            pltpu.VMEM((C, D), f32),
            pltpu.VMEM((C, D), f32),
            pltpu.VMEM((C, D), f32),
            pltpu.VMEM((ND0,), f32),
        ] + [pltpu.SemaphoreType.DMA] * 7,
        compiler_params=_SC_PARAMS,
        name="sc_g0_agg",
    )(x0, src0, dst0)

    xs0 = pl.pallas_call(
        _lin_body,
        grid=(5,),
        in_specs=[pl.BlockSpec((2000, D), lambda i: (i, 0)),
                  pl.BlockSpec((D, D), lambda i: (0, 0))],
        out_specs=pl.BlockSpec((2000, D), lambda i: (i, 0)),
        out_shape=jax.ShapeDtypeStruct((N0, D), f32),
    )(x0, W_self0)
    xs1 = pl.pallas_call(
        _lin_body,
        out_shape=jax.ShapeDtypeStruct((N1, D), f32),
    )(x1, W_self1)

    h0, s0, z0 = pl.pallas_call(
        _tc2a_body,
        out_shape=[jax.ShapeDtypeStruct((N0, D), f32),
                   jax.ShapeDtypeStruct((N0, D), f32),
                   jax.ShapeDtypeStruct((N0, D), f32)],
        compiler_params=pltpu.CompilerParams(vmem_limit_bytes=100 * 1024 * 1024),
    )(p0, d0p, xs0, W_neigh0, W_self_i, W_neigh_i)

    h1, s1, z1 = pl.pallas_call(
        _tc2b_body,
        out_shape=[jax.ShapeDtypeStruct((N1, D), f32),
                   jax.ShapeDtypeStruct((N1, D), f32),
                   jax.ShapeDtypeStruct((N1, D), f32)],
    )(p1, d1p, xs1, W_neigh1, W_self_i, W_neigh_i)

    g, c2, cntp = pl.kernel(
        _sc_inter_body,
        out_type=[
            jax.ShapeDtypeStruct((N0, D), f32),
            jax.ShapeDtypeStruct((NC, N1P, D), f32),
            jax.ShapeDtypeStruct((NW, N1P), f32),
        ],
        mesh=mesh,
        scratch_types=[
            pltpu.VMEM_SHARED((N1P, D), f32),
            pltpu.VMEM_SHARED((N1, D), f32),
            pltpu.VMEM((JI, CI), i32),
            pltpu.VMEM((CI, D), f32),
            pltpu.VMEM((CI, D), f32),
            pltpu.VMEM((CI, D), f32),
            pltpu.VMEM((CI, D), f32),
            pltpu.VMEM((CI, D), f32),
            pltpu.VMEM((CI, D), f32),
            pltpu.VMEM((CI, D), f32),
            pltpu.VMEM((CI, D), f32),
            pltpu.VMEM((N1P,), f32),
        ] + [pltpu.SemaphoreType.DMA] * 8,
        compiler_params=_SC_PARAMS,
        name="sc_inter",
    )(z0, z1, clc)

    out0 = pl.pallas_call(
        _tc3a_body,
        grid=(5,),
        in_specs=[pl.BlockSpec((2000, D), lambda i: (i, 0))] * 4,
        out_specs=pl.BlockSpec((2000, D), lambda i: (i, 0)),
        out_shape=jax.ShapeDtypeStruct((N0, D), f32),
    )(x0, h0, s0, g)

    out1 = pl.pallas_call(
        _tc3b_body,
        out_shape=jax.ShapeDtypeStruct((N1, D), f32),
    )(x1, h1, s1, c2, cntp)

    return (out0, out1)


# speedup vs baseline: 2.8183x; 1.0643x over previous
import jax
import jax.numpy as jnp
from jax import lax
from jax.experimental import pallas as pl
from jax.experimental.pallas import tpu as pltpu
from jax.experimental.pallas import tpu_sc as plsc

N0, E0 = 10000, 320000
N1, E1 = 2500, 40000
D = 128
NC, NS = 2, 16
NW = NC * NS

ND0 = 10112
N1P = 2560
EPT1 = 1280
C = 80
B0, J0 = 5, 25
J1 = 16
JI, CI = 4, 80

_SC_PARAMS = pltpu.CompilerParams(needs_layout_passes=False,
                                  use_tc_tiling_on_sc=False)


def _zero_vec16():
    return jnp.zeros((16,), jnp.float32)


def _zero_2d(ref, rows):
    def _zrow(r, _):
        for k in range(8):
            ref[r, pl.ds(k * 16, 16)] = _zero_vec16()
        return 0
    lax.fori_loop(0, rows, _zrow, 0)


def _zero_1d(ref, n):
    def _z(i, _):
        ref[pl.ds(i * 16, 16)] = _zero_vec16()
        return 0
    lax.fori_loop(0, n // 16, _z, 0)


def _sc_g0_body(x0h, se0h, de0h,
                p0o, d0o,
                acc0, sev0, sev1, dev0, dev1, b0, b1, b2, deg0,
                gs0, gs1, gs2, ss0, ss1, ss2, isem):
    cid = lax.axis_index("c")
    sid = lax.axis_index("s")
    wid = cid * NS + sid

    sev = [sev0, sev1]
    dev = [dev0, dev1]
    bufs = [b0, b1, b2]
    gsem = [gs0, gs1, gs2]
    ssem = [ss0, ss1, ss2]

    _zero_2d(b0, C)
    _zero_1d(deg0, ND0)
    for t in range(7):
        pltpu.sync_copy(b0, acc0.at[pl.ds(sid * 625 + t * 80, 80)])
    pltpu.sync_copy(b0.at[pl.ds(0, 65)], acc0.at[pl.ds(sid * 625 + 560, 65)])
    plsc.subcore_barrier()

    pltpu.sync_copy(se0h.at[wid, 0], sev[0])
    pltpu.sync_copy(de0h.at[wid, 0], dev[0])

    ones16 = jnp.ones((16,), jnp.float32)
    gd = [None, None, None]
    sd = [None, None, None]
    idxd = []
    gd[0] = pltpu.async_copy(x0h.at[sev[0].at[0]], bufs[0], gsem[0])
    gd[1] = pltpu.async_copy(x0h.at[sev[0].at[1]], bufs[1], gsem[1])

    for b in range(B0):
        ib = b % 2
        if b + 1 < B0:
            jprev = b * J0 - 1
            if jprev >= 0 and sd[jprev % 3] is not None:
                sd[jprev % 3].wait()
                sd[jprev % 3] = None
            nb = 1 - ib
            idxd = [pltpu.async_copy(se0h.at[wid, b + 1], sev[nb], isem),
                    pltpu.async_copy(de0h.at[wid, b + 1], dev[nb], isem)]
        for r in range(J0):
            j = b * J0 + r
            k = j % 3
            gd[k].wait()
            gd[k] = None
            k2 = (j + 2) % 3
            if sd[k2] is not None:
                sd[k2].wait()
                sd[k2] = None
            jj = j + 2
            if jj < B0 * J0:
                bb, rr = divmod(jj, J0)
                if bb != b and idxd:
                    for dsc in idxd:
                        dsc.wait()
                    idxd = []
                gd[k2] = pltpu.async_copy(
                    x0h.at[sev[bb % 2].at[rr]], bufs[k2], gsem[k2])
            sd[k] = pltpu.async_copy(
                bufs[k], acc0.at[dev[ib].at[r]], ssem[k], add=True)

        def _h(t, _, ib=ib):
            r = t // 5
            g = t % 5
            plsc.addupdate_scatter(deg0, [dev[ib][r, pl.ds(g * 16, 16)]],
                                   ones16)
            return 0
        lax.fori_loop(0, J0 * 5, _h, 0)

    for k in range(3):
        if sd[k] is not None:
            sd[k].wait()

    plsc.subcore_barrier()
    pltpu.sync_copy(acc0.at[pl.ds(sid * 625, 625)],
                    p0o.at[cid, pl.ds(sid * 625, 625)])
    pltpu.sync_copy(deg0, d0o.at[wid])


def _sc_g1_body(x1h, se1h, de1h,
                p1o, d1o,
                acc1, x1s, se1v, de1v, bufa, bufb, deg1,
                gsa, gsb, ssa, ssb):
    cid = lax.axis_index("c")
    sid = lax.axis_index("s")
    wid = cid * NS + sid

    bufs = [bufa, bufb]
    gsem = [gsa, gsb]
    ssem = [ssa, ssb]

    @pl.when(sid < 4)
    def _():
        pltpu.sync_copy(x1h.at[pl.ds(sid * 625, 625)],
                        x1s.at[pl.ds(sid * 625, 625)])

    _zero_2d(bufa, C)
    _zero_1d(deg1, N1P)
    pltpu.sync_copy(bufa, acc1.at[pl.ds(sid * 160, 80)])
    pltpu.sync_copy(bufa, acc1.at[pl.ds(sid * 160 + 80, 80)])
    pltpu.sync_copy(se1h.at[wid], se1v)
    pltpu.sync_copy(de1h.at[wid], de1v)
    plsc.subcore_barrier()

    gd = [None, None]
    sd = [None, None]
    gd[0] = pltpu.async_copy(x1s.at[se1v.at[0]], bufs[0], gsem[0])
    for r in range(J1):
        k = r % 2
        gd[k].wait()
        gd[k] = None
        if sd[1 - k] is not None:
            sd[1 - k].wait()
            sd[1 - k] = None
        if r + 1 < J1:
            gd[1 - k] = pltpu.async_copy(
                x1s.at[se1v.at[r + 1]], bufs[1 - k], gsem[1 - k])
        sd[k] = pltpu.async_copy(
            bufs[k], acc1.at[de1v.at[r]], ssem[k], add=True)

    ones16 = jnp.ones((16,), jnp.float32)

    def _h1(t, _):
        r = t // 5
        g = t % 5
        plsc.addupdate_scatter(deg1, [de1v[r, pl.ds(g * 16, 16)]], ones16)
        return 0
    lax.fori_loop(0, J1 * 5, _h1, 0)

    for k in range(2):
        if sd[k] is not None:
            sd[k].wait()

    plsc.subcore_barrier()
    pltpu.sync_copy(acc1.at[pl.ds(sid * 160, 160)],
                    p1o.at[cid, pl.ds(sid * 160, 160)])
    pltpu.sync_copy(deg1, d1o.at[wid])


def _sc_inter_body(z0h, z1h, clch, go, c2o, cnto,
                   accc, z1s, clv, r0, r1, r2, r3, g0b, g1b, g2b, g3b, cnt,
                   rs0, rs1, rs2, rs3, gs0, gs1, gs2, gs3):
    cid = lax.axis_index("c")
    sid = lax.axis_index("s")
    wid = cid * NS + sid

    rbufs = [r0, r1, r2, r3]
    gbufs = [g0b, g1b, g2b, g3b]
    rsem = [rs0, rs1, rs2, rs3]
    gsem = [gs0, gs1, gs2, gs3]

    @pl.when(sid < 4)
    def _():
        pltpu.sync_copy(z1h.at[pl.ds(sid * 625, 625)],
                        z1s.at[pl.ds(sid * 625, 625)])

    _zero_2d(r0, CI)
    _zero_1d(cnt, N1P)
    pltpu.sync_copy(r0, accc.at[pl.ds(sid * 160, 80)])
    pltpu.sync_copy(r0, accc.at[pl.ds(sid * 160 + 80, 80)])
    pltpu.sync_copy(clch.at[wid], clv)
    plsc.subcore_barrier()

    ones16 = jnp.ones((16,), jnp.float32)
    base0 = wid * (JI * CI)

    for j in range(JI):
        valid = base0 + j * CI < N0

        @pl.when(valid)
        def _(j=j):
            pltpu.async_copy(z0h.at[pl.ds(base0 + j * CI, CI)],
                             rbufs[j], rsem[j])
            pltpu.async_copy(z1s.at[clv.at[j]], gbufs[j], gsem[j])

    for j in range(JI):
        valid = base0 + j * CI < N0

        @pl.when(valid)
        def _(j=j):
            pltpu.make_async_copy(z0h.at[pl.ds(base0 + j * CI, CI)],
                                  rbufs[j], rsem[j]).wait()
            pltpu.sync_copy(rbufs[j], accc.at[clv.at[j]], add=True)
            for g in range(5):
                plsc.addupdate_scatter(cnt, [clv[j, pl.ds(g * 16, 16)]],
                                       ones16)

    for j in range(JI):
        valid = base0 + j * CI < N0

        @pl.when(valid)
        def _(j=j):
            pltpu.make_async_copy(z1s.at[clv.at[j]], gbufs[j],
                                  gsem[j]).wait()
            pltpu.sync_copy(gbufs[j], go.at[pl.ds(base0 + j * CI, CI)])

    plsc.subcore_barrier()
    pltpu.sync_copy(accc.at[pl.ds(sid * 160, 160)],
                    c2o.at[cid, pl.ds(sid * 160, 160)])
    pltpu.sync_copy(cnt, cnto.at[wid])


def _mm(a, w):
    return lax.dot_general(a, w, (((1,), (0,)), ((), ())),
                           preferred_element_type=jnp.float32)


def _lin_body(x_ref, w_ref, o_ref):
    o_ref[...] = _mm(x_ref[...], w_ref[...])


def _tc2a_body(p0_ref, d0_ref, xs_ref, wn_ref, wsi_ref, wni_ref,
               h_ref, s_ref, z_ref):
    a = p0_ref[0] + p0_ref[1]
    deg = jnp.sum(d0_ref[...], axis=0)[:N0]
    m = a / jnp.maximum(deg, 1.0)[:, None]
    h = jnp.maximum(xs_ref[...] + _mm(m, wn_ref[...]), 0.0)
    h_ref[...] = h
    s_ref[...] = _mm(h, wsi_ref[...])
    z_ref[...] = _mm(h, wni_ref[...])


def _tc2b_body(p1_ref, d1_ref, xs_ref, wn_ref, wsi_ref, wni_ref,
               h_ref, s_ref, z_ref):
    a = (p1_ref[0] + p1_ref[1])[:N1]
    deg = jnp.sum(d1_ref[...], axis=0)[:N1]
    m = a / jnp.maximum(deg, 1.0)[:, None]
    h = jnp.maximum(xs_ref[...] + _mm(m, wn_ref[...]), 0.0)
    h_ref[...] = h
    s_ref[...] = _mm(h, wsi_ref[...])
    z_ref[...] = _mm(h, wni_ref[...])


def _tc3a_body(x_ref, h_ref, s_ref, g_ref, o_ref):
    nz = jnp.maximum(s_ref[...] + g_ref[...], 0.0)
    o_ref[...] = x_ref[...] + 0.5 * (h_ref[...] + nz)


def _tc3b_body(x_ref, h_ref, s_ref, c2_ref, cnt_ref, o_ref):
    cnt = jnp.sum(cnt_ref[...], axis=0)[:N1]
    cs = (c2_ref[0] + c2_ref[1])[:N1]
    nz = jnp.maximum(s_ref[...] + cs / jnp.maximum(cnt, 1.0)[:, None], 0.0)
    o_ref[...] = x_ref[...] + 0.5 * (h_ref[...] + nz)


def kernel(x0, x1, edge_index0, edge_index1, inter_edge_index,
           W_self0, W_neigh0, W_self1, W_neigh1, W_self_i, W_neigh_i):
    f32 = jnp.float32
    i32 = jnp.int32

    src0 = edge_index0[0].reshape(NW, B0, J0, C)
    dst0 = edge_index0[1].reshape(NW, B0, J0, C)
    pad1 = EPT1 - E1 // NW
    s1 = jnp.pad(edge_index1[0].reshape(NW, E1 // NW), ((0, 0), (0, pad1)))
    d1junk = N1 + (jnp.arange(pad1, dtype=i32) % (N1P - N1))
    d1 = jnp.concatenate(
        [edge_index1[1].reshape(NW, E1 // NW),
         jnp.broadcast_to(d1junk, (NW, pad1))], axis=1)
    src1 = s1.reshape(NW, J1, C)
    dst1 = d1.reshape(NW, J1, C)
    cluster = inter_edge_index[1, :N0] - N0
    clc = jnp.pad(cluster, (0, NW * JI * CI - N0),
                  constant_values=N1).reshape(NW, JI, CI)

    mesh = plsc.VectorSubcoreMesh(core_axis_name="c", subcore_axis_name="s")

    p1, d1p = pl.kernel(
        _sc_g1_body,
        out_type=[
            jax.ShapeDtypeStruct((NC, N1P, D), f32),
            jax.ShapeDtypeStruct((NW, N1P), f32),
        ],
        mesh=mesh,
        scratch_types=[
            pltpu.VMEM_SHARED((N1P, D), f32),
            pltpu.VMEM_SHARED((N1, D), f32),
            pltpu.VMEM((J1, C), i32),
            pltpu.VMEM((J1, C), i32),
            pltpu.VMEM((C, D), f32),
            pltpu.VMEM((C, D), f32),
            pltpu.VMEM((N1P,), f32),
        ] + [pltpu.SemaphoreType.DMA] * 4,
        compiler_params=_SC_PARAMS,
        name="sc_g1_agg",
    )(x1, src1, dst1)

    x0g, _ = lax.optimization_barrier((x0, d1p))
    p0, d0p = pl.kernel(
        _sc_g0_body,
        out_type=[
            jax.ShapeDtypeStruct((NC, N0, D), f32),
            jax.ShapeDtypeStruct((NW, ND0), f32),
        ],
        mesh=mesh,
        scratch_types=[
            pltpu.VMEM_SHARED((N0, D), f32),
            pltpu.VMEM((J0, C), i32),
            pltpu.VMEM((J0, C), i32),
            pltpu.VMEM((J0, C), i32),
            pltpu.VMEM((J0, C), i32),
            pltpu.VMEM((C, D), f32),
            pltpu.VMEM((C, D), f32),
            pltpu.VMEM((C, D), f32),
            pltpu.VMEM((ND0,), f32),
        ] + [pltpu.SemaphoreType.DMA] * 7,
        compiler_params=_SC_PARAMS,
        name="sc_g0_agg",
    )(x0g, src0, dst0)

    xs0 = pl.pallas_call(
        _lin_body,
        grid=(5,),
        in_specs=[pl.BlockSpec((2000, D), lambda i: (i, 0)),
                  pl.BlockSpec((D, D), lambda i: (0, 0))],
        out_specs=pl.BlockSpec((2000, D), lambda i: (i, 0)),
        out_shape=jax.ShapeDtypeStruct((N0, D), f32),
    )(x0, W_self0)
    xs1 = pl.pallas_call(
        _lin_body,
        out_shape=jax.ShapeDtypeStruct((N1, D), f32),
    )(x1, W_self1)

    h0, s0, z0 = pl.pallas_call(
        _tc2a_body,
        out_shape=[jax.ShapeDtypeStruct((N0, D), f32),
                   jax.ShapeDtypeStruct((N0, D), f32),
                   jax.ShapeDtypeStruct((N0, D), f32)],
        compiler_params=pltpu.CompilerParams(vmem_limit_bytes=100 * 1024 * 1024),
    )(p0, d0p, xs0, W_neigh0, W_self_i, W_neigh_i)

    h1, s1, z1 = pl.pallas_call(
        _tc2b_body,
        out_shape=[jax.ShapeDtypeStruct((N1, D), f32),
                   jax.ShapeDtypeStruct((N1, D), f32),
                   jax.ShapeDtypeStruct((N1, D), f32)],
    )(p1, d1p, xs1, W_neigh1, W_self_i, W_neigh_i)

    g, c2, cntp = pl.kernel(
        _sc_inter_body,
        out_type=[
            jax.ShapeDtypeStruct((N0, D), f32),
            jax.ShapeDtypeStruct((NC, N1P, D), f32),
            jax.ShapeDtypeStruct((NW, N1P), f32),
        ],
        mesh=mesh,
        scratch_types=[
            pltpu.VMEM_SHARED((N1P, D), f32),
            pltpu.VMEM_SHARED((N1, D), f32),
            pltpu.VMEM((JI, CI), i32),
            pltpu.VMEM((CI, D), f32),
            pltpu.VMEM((CI, D), f32),
            pltpu.VMEM((CI, D), f32),
            pltpu.VMEM((CI, D), f32),
            pltpu.VMEM((CI, D), f32),
            pltpu.VMEM((CI, D), f32),
            pltpu.VMEM((CI, D), f32),
            pltpu.VMEM((CI, D), f32),
            pltpu.VMEM((N1P,), f32),
        ] + [pltpu.SemaphoreType.DMA] * 8,
        compiler_params=_SC_PARAMS,
        name="sc_inter",
    )(z0, z1, clc)

    out0 = pl.pallas_call(
        _tc3a_body,
        grid=(5,),
        in_specs=[pl.BlockSpec((2000, D), lambda i: (i, 0))] * 4,
        out_specs=pl.BlockSpec((2000, D), lambda i: (i, 0)),
        out_shape=jax.ShapeDtypeStruct((N0, D), f32),
    )(x0, h0, s0, g)

    out1 = pl.pallas_call(
        _tc3b_body,
        out_shape=jax.ShapeDtypeStruct((N1, D), f32),
    )(x1, h1, s1, c2, cntp)

    return (out0, out1)
